# baseline (speedup 1.0000x reference)
import jax
import jax.numpy as jnp
from jax.experimental import pallas as pl
from jax.experimental.pallas import tpu as pltpu


def _bf16(v):
    return v.astype(jnp.bfloat16)


def _xw_kernel(x_ref, w_ref, o_ref):
    o_ref[...] = _bf16(jnp.dot(_bf16(x_ref[...]), w_ref[...],
                               preferred_element_type=jnp.float32))


def _first_layer_kernel(adj_ref, s_ref, b_ref, w_ref, o_ref, adjb_ref):
    adjb = _bf16(adj_ref[...])
    adjb_ref[...] = adjb
    acc = jnp.dot(adjb, s_ref[...], preferred_element_type=jnp.float32)
    h = _bf16(jnp.maximum(acc + b_ref[...], 0.0))
    o_ref[...] = _bf16(jnp.dot(h, w_ref[...],
                               preferred_element_type=jnp.float32))


def _make_l23_kernel(bm, nblk, rb):
    def body(adj_ref, s2_ref, b2_ref, b3_ref, w3_ref, out_ref,
             res_ref, s3_ref):
        l = pl.program_id(0)
        i = pl.program_id(1)
        cnt = ((i + 1) * rb) // nblk
        is_res = cnt > (i * rb) // nblk
        r_idx = cnt - 1

        @pl.when(l == 0)
        def _():
            a = adj_ref[...]

            @pl.when(i < rb)
            def _():
                res_ref[pl.ds(i * bm, bm), :] = a

            acc = jnp.dot(a, s2_ref[...], preferred_element_type=jnp.float32)
            h = _bf16(jnp.maximum(acc + b2_ref[...], 0.0))
            s3_ref[pl.ds(i * bm, bm), :] = _bf16(
                jnp.dot(h, w3_ref[...], preferred_element_type=jnp.float32))

        @pl.when(l == 1)
        def _():
            def compute(src):
                acc = jnp.dot(src, s3_ref[...],
                              preferred_element_type=jnp.float32)
                h = acc + b3_ref[...]
                m = jnp.max(h, axis=1, keepdims=True)
                lse = jnp.log(jnp.sum(jnp.exp(h - m), axis=1,
                                      keepdims=True)) + m
                out_ref[...] = h - lse

            @pl.when(is_res)
            def _():
                compute(res_ref[pl.ds(r_idx * bm, bm), :])

            @pl.when(jnp.logical_not(is_res))
            def _():
                compute(adj_ref[...])

    return body


def _row_block(n, target):
    for bm in (target, 400, 200, 80, 40, 8):
        if bm <= target and n % bm == 0:
            return bm
    return n


@jax.jit
def kernel(x, adj, W1, b1, W2, b2, W3, b3):
    n = adj.shape[0]
    nh = W2.shape[0]
    nc = W3.shape[1]
    w1, w2, w3 = _bf16(W1), _bf16(W2), _bf16(W3)
    s1 = pl.pallas_call(
        _xw_kernel,
        out_shape=jax.ShapeDtypeStruct((n, nh), jnp.bfloat16),
    )(x, w1)
    b1r = b1.reshape(1, -1)
    b2r = b2.reshape(1, -1)
    b3r = b3.reshape(1, -1)

    bm1 = _row_block(n, 400)
    s2, adj_bf = pl.pallas_call(
        _first_layer_kernel,
        grid=(n // bm1,),
        in_specs=[
            pl.BlockSpec((bm1, n), lambda i: (i, 0)),
            pl.BlockSpec((n, nh), lambda i: (0, 0)),
            pl.BlockSpec((1, nh), lambda i: (0, 0)),
            pl.BlockSpec((nh, nh), lambda i: (0, 0)),
        ],
        out_specs=[
            pl.BlockSpec((bm1, nh), lambda i: (i, 0)),
            pl.BlockSpec((bm1, n), lambda i: (i, 0)),
        ],
        out_shape=[
            jax.ShapeDtypeStruct((n, nh), jnp.bfloat16),
            jax.ShapeDtypeStruct((n, n), jnp.bfloat16),
        ],
    )(adj, s1, b1r, w2)

    bm = 80 if n % 80 == 0 else _row_block(n, 400)
    nblk = n // bm
    rb = max(1, min(34, nblk - 2))

    def res_blk(idx):
        c = ((idx + 1) * rb) // nblk
        return jnp.where(c > (idx * rb) // nblk, c - 1, rb + idx - c)

    out = pl.pallas_call(
        _make_l23_kernel(bm, nblk, rb),
        grid=(2, nblk),
        in_specs=[
            pl.BlockSpec((bm, n),
                         lambda l, i: (jnp.where(
                             l == 0, i,
                             rb + i - ((i + 1) * rb) // nblk), 0)),
            pl.BlockSpec((n, nh), lambda l, i: (0, 0)),
            pl.BlockSpec((1, nh), lambda l, i: (0, 0)),
            pl.BlockSpec((1, nc), lambda l, i: (0, 0)),
            pl.BlockSpec((nh, nc), lambda l, i: (0, 0)),
        ],
        out_specs=pl.BlockSpec((bm, nc),
                               lambda l, i: (jnp.where(l == 1, res_blk(i), rb),
                                             0)),
        out_shape=jax.ShapeDtypeStruct((n, nc), jnp.float32),
        scratch_shapes=[
            pltpu.VMEM((rb * bm, n), jnp.bfloat16),
            pltpu.VMEM((n, nc), jnp.bfloat16),
        ],
        compiler_params=pltpu.CompilerParams(
            dimension_semantics=("arbitrary", "arbitrary"),
            vmem_limit_bytes=64 * 1024 * 1024,
        ),
    )(adj_bf, s2, b2r, b3r, w3)
    return out


# speedup vs baseline: 1.4143x; 1.4143x over previous
import jax
import jax.numpy as jnp
from jax.experimental import pallas as pl


def _bf16(v):
    return v.astype(jnp.bfloat16)


def _xw_kernel(x_ref, w_ref, o_ref):
    o_ref[...] = _bf16(jnp.dot(_bf16(x_ref[...]), w_ref[...],
                               preferred_element_type=jnp.float32))


def _first_layer_kernel(adj_ref, s_ref, b_ref, w_ref, o_ref, adjb_ref):
    adjb = _bf16(adj_ref[...])
    adjb_ref[...] = adjb
    acc = jnp.dot(adjb, s_ref[...], preferred_element_type=jnp.float32)
    h = _bf16(jnp.maximum(acc + b_ref[...], 0.0))
    o_ref[...] = _bf16(jnp.dot(h, w_ref[...],
                               preferred_element_type=jnp.float32))


def _mid_layer_kernel(adj_ref, s_ref, b_ref, w_ref, o_ref):
    acc = jnp.dot(adj_ref[...], s_ref[...], preferred_element_type=jnp.float32)
    h = _bf16(jnp.maximum(acc + b_ref[...], 0.0))
    o_ref[...] = _bf16(jnp.dot(h, w_ref[...],
                               preferred_element_type=jnp.float32))


def _last_layer_kernel(adj_ref, s_ref, b_ref, o_ref):
    acc = jnp.dot(adj_ref[...], s_ref[...], preferred_element_type=jnp.float32)
    h = acc + b_ref[...]
    m = jnp.max(h, axis=1, keepdims=True)
    lse = jnp.log(jnp.sum(jnp.exp(h - m), axis=1, keepdims=True)) + m
    o_ref[...] = h - lse


def _row_block(n, target):
    for bm in (target, 400, 200, 80, 40, 8):
        if bm <= target and n % bm == 0:
            return bm
    return n


def _layer_call(body, adj, s, b, extra, out_cols, out_dtype, bm_target,
                emit_adj_bf16=False):
    n = adj.shape[0]
    bm = _row_block(n, bm_target)
    grid = (n // bm,)
    k = s.shape[1]
    in_specs = [
        pl.BlockSpec((bm, n), lambda i: (i, 0)),
        pl.BlockSpec((n, k), lambda i: (0, 0)),
        pl.BlockSpec((1, b.shape[1]), lambda i: (0, 0)),
    ]
    args = [adj, s, b]
    if extra is not None:
        in_specs.append(pl.BlockSpec(extra.shape, lambda i: (0, 0)))
        args.append(extra)
    out_specs = pl.BlockSpec((bm, out_cols), lambda i: (i, 0))
    out_shape = jax.ShapeDtypeStruct((n, out_cols), out_dtype)
    if emit_adj_bf16:
        out_specs = [out_specs, pl.BlockSpec((bm, n), lambda i: (i, 0))]
        out_shape = [out_shape, jax.ShapeDtypeStruct((n, n), jnp.bfloat16)]
    return pl.pallas_call(
        body,
        grid=grid,
        in_specs=in_specs,
        out_specs=out_specs,
        out_shape=out_shape,
    )(*args)


@jax.jit
def kernel(x, adj, W1, b1, W2, b2, W3, b3):
    n = adj.shape[0]
    w1, w2, w3 = _bf16(W1), _bf16(W2), _bf16(W3)
    s1 = pl.pallas_call(
        _xw_kernel,
        out_shape=jax.ShapeDtypeStruct((n, W1.shape[1]), jnp.bfloat16),
    )(x, w1)
    b1r = b1.reshape(1, -1)
    b2r = b2.reshape(1, -1)
    b3r = b3.reshape(1, -1)
    s2, adj_bf = _layer_call(_first_layer_kernel, adj, s1, b1r, w2,
                             W2.shape[1], jnp.bfloat16, 400,
                             emit_adj_bf16=True)
    s3 = _layer_call(_mid_layer_kernel, adj_bf, s2, b2r, w3,
                     W3.shape[1], jnp.bfloat16, 400)
    out = _layer_call(_last_layer_kernel, adj_bf, s3, b3r, None,
                      W3.shape[1], jnp.float32, 400)
    return out
